# Initial kernel scaffold; baseline (speedup 1.0000x reference)
import jax
import jax.numpy as jnp
from jax.experimental import pallas as pl


def kernel(x, edge_index, W1, b1, W2, b2, W3, b3):
    raise NotImplementedError("write your pallas kernel here")


# speedup vs baseline: 5.6858x; 5.6858x over previous
import functools

import jax
import jax.numpy as jnp
from jax import lax
from jax.experimental import pallas as pl
from jax.experimental.pallas import tpu as pltpu
from jax.experimental.pallas import tpu_sc as plsc

N = 10000
NP = 10112
NC = 2
NS = 16
NW = NC * NS
C = 128
K = 80
EPW = K * C
EP = NW * EPW
RPT = NP // NS
F32 = jnp.float32


def _sc_aggregate(g, src3, dst3, d):
    mesh = plsc.VectorSubcoreMesh(core_axis_name="c", subcore_axis_name="s")

    @functools.partial(
        pl.kernel,
        out_type=jax.ShapeDtypeStruct((NC, NP, d), F32),
        mesh=mesh,
        scratch_types=[
            pltpu.VMEM((K, C), jnp.int32),
            pltpu.VMEM((K, C), jnp.int32),
            pltpu.VMEM((C, d), F32),
            pltpu.VMEM_SHARED((NP, d), F32),
            pltpu.SemaphoreType.DMA,
        ],
    )
    def k(g_hbm, src_hbm, dst_hbm, out_hbm, src_v, dst_v, buf, acc, sem):
        c = lax.axis_index("c")
        s = lax.axis_index("s")
        wid = s * NC + c
        pltpu.sync_copy(src_hbm.at[wid], src_v)
        pltpu.sync_copy(dst_hbm.at[wid], dst_v)
        r0 = s * RPT
        chunks = [(i * C, C) for i in range(RPT // C)] + (
            [((RPT // C) * C, RPT % C)] if RPT % C else [])
        for off, sz in chunks:
            pltpu.sync_copy(g_hbm.at[pl.ds(r0 + off, sz)], buf.at[pl.ds(0, sz)])
            pltpu.sync_copy(buf.at[pl.ds(0, sz)], acc.at[pl.ds(r0 + off, sz)])
        plsc.subcore_barrier()

        def body(j, _):
            pltpu.async_copy(g_hbm.at[src_v.at[j]], buf, sem).wait()
            pltpu.sync_copy(buf, acc.at[dst_v.at[j]], add=True)
            return 0

        lax.fori_loop(0, K, body, 0)
        plsc.subcore_barrier()
        for off, sz in chunks:
            pltpu.sync_copy(acc.at[pl.ds(r0 + off, sz)], buf.at[pl.ds(0, sz)])
            pltpu.sync_copy(buf.at[pl.ds(0, sz)], out_hbm.at[c, pl.ds(r0 + off, sz)])

    return k(g, src3, dst3)


_R = 2528


def _dot(a, w):
    return lax.dot_general(a, w, (((1,), (0,)), ((), ())),
                           precision=lax.Precision.HIGHEST,
                           preferred_element_type=F32)


def _tc_prep(deg0, deg1, xpad, w1):
    def body(d0, d1, x, w, g_out, dv_out):
        dsum = d0[...][:, :1] + d1[...][:, :1] - 1.0
        dv = lax.rsqrt(dsum)
        dv_out[...] = dv
        g_out[...] = dv * _dot(x[...], w[...])

    return pl.pallas_call(
        body,
        grid=(NP // _R,),
        in_specs=[
            pl.BlockSpec((_R, 128), lambda i: (i, 0)),
            pl.BlockSpec((_R, 128), lambda i: (i, 0)),
            pl.BlockSpec((_R, 128), lambda i: (i, 0)),
            pl.BlockSpec((128, 128), lambda i: (0, 0)),
        ],
        out_specs=(
            pl.BlockSpec((_R, 128), lambda i: (i, 0)),
            pl.BlockSpec((_R, 1), lambda i: (i, 0)),
        ),
        out_shape=(
            jax.ShapeDtypeStruct((NP, 128), F32),
            jax.ShapeDtypeStruct((NP, 1), F32),
        ),
    )(deg0, deg1, xpad, w1)


def _tc_mid(p0, p1, g, dinv, b, w, d_out):
    def body(p0r, p1r, gr, dvr, br, wr, out):
        dv = dvr[...]
        z = dv * (p0r[...] + p1r[...] - gr[...]) + br[...]
        h = jnp.maximum(z, 0.0)
        out[...] = dv * _dot(h, wr[...])

    return pl.pallas_call(
        body,
        grid=(NP // _R,),
        in_specs=[
            pl.BlockSpec((_R, 128), lambda i: (i, 0)),
            pl.BlockSpec((_R, 128), lambda i: (i, 0)),
            pl.BlockSpec((_R, 128), lambda i: (i, 0)),
            pl.BlockSpec((_R, 1), lambda i: (i, 0)),
            pl.BlockSpec((1, 128), lambda i: (0, 0)),
            pl.BlockSpec((128, d_out), lambda i: (0, 0)),
        ],
        out_specs=pl.BlockSpec((_R, d_out), lambda i: (i, 0)),
        out_shape=jax.ShapeDtypeStruct((NP, d_out), F32),
    )(p0, p1, g, dinv, b, w)


def _tc_scale_relu(p0, p1, g, dinv, b):
    def body(p0r, p1r, gr, dvr, br, out):
        dv = dvr[...]
        z = dv * (p0r[...] + p1r[...] - gr[...]) + br[...]
        out[...] = dv * jnp.maximum(z, 0.0)

    return pl.pallas_call(
        body,
        grid=(NP // _R,),
        in_specs=[
            pl.BlockSpec((_R, 128), lambda i: (i, 0)),
            pl.BlockSpec((_R, 128), lambda i: (i, 0)),
            pl.BlockSpec((_R, 128), lambda i: (i, 0)),
            pl.BlockSpec((_R, 1), lambda i: (i, 0)),
            pl.BlockSpec((1, 128), lambda i: (0, 0)),
        ],
        out_specs=pl.BlockSpec((_R, 128), lambda i: (i, 0)),
        out_shape=jax.ShapeDtypeStruct((NP, 128), F32),
    )(p0, p1, g, dinv, b)


def _tc_final(p0, p1, u, dinv, w, b):
    def body(p0r, p1r, ur, dvr, wr, br, out):
        agg = dvr[...] * (p0r[...] + p1r[...] - ur[...])
        z = _dot(agg, wr[...]) + br[...]
        m = jnp.max(z, axis=1, keepdims=True)
        e = jnp.exp(z - m)
        lse = jnp.log(jnp.sum(e, axis=1, keepdims=True)) + m
        out[...] = z - lse

    return pl.pallas_call(
        body,
        grid=(NP // _R,),
        in_specs=[
            pl.BlockSpec((_R, 128), lambda i: (i, 0)),
            pl.BlockSpec((_R, 128), lambda i: (i, 0)),
            pl.BlockSpec((_R, 128), lambda i: (i, 0)),
            pl.BlockSpec((_R, 1), lambda i: (i, 0)),
            pl.BlockSpec((128, 16), lambda i: (0, 0)),
            pl.BlockSpec((1, 16), lambda i: (0, 0)),
        ],
        out_specs=pl.BlockSpec((_R, 16), lambda i: (i, 0)),
        out_shape=jax.ShapeDtypeStruct((NP, 16), F32),
    )(p0, p1, u, dinv, w, b)


def kernel(x, edge_index, W1, b1, W2, b2, W3, b3):
    src = edge_index[0].astype(jnp.int32)
    dst = edge_index[1].astype(jnp.int32)
    e = src.shape[0]
    pad = EP - e
    src3 = jnp.concatenate([src, jnp.full((pad,), N, jnp.int32)]).reshape(NW, K, C)
    dst3 = jnp.concatenate([dst, jnp.full((pad,), N, jnp.int32)]).reshape(NW, K, C)
    xpad = jnp.zeros((NP, x.shape[1]), F32).at[:N].set(x)

    degp = _sc_aggregate(jnp.ones((NP, 128), F32), src3, dst3, 128)
    g1, dinv = _tc_prep(degp[0], degp[1], xpad, W1)

    p1 = _sc_aggregate(g1, src3, dst3, 128)
    g2 = _tc_mid(p1[0], p1[1], g1, dinv, b1.reshape(1, -1), W2, 128)

    p2 = _sc_aggregate(g2, src3, dst3, 128)
    u = _tc_scale_relu(p2[0], p2[1], g2, dinv, b2.reshape(1, -1))

    p3 = _sc_aggregate(u, src3, dst3, 128)
    out = _tc_final(p3[0], p3[1], u, dinv, W3, b3.reshape(1, -1))
    return out[:N]


# speedup vs baseline: 6.1240x; 1.0771x over previous
import functools

import jax
import jax.numpy as jnp
from jax import lax
from jax.experimental import pallas as pl
from jax.experimental.pallas import tpu as pltpu
from jax.experimental.pallas import tpu_sc as plsc

N = 10000
NP = 10112
NC = 2
NS = 16
NW = NC * NS
C = 128
K = 80
KC = 40
EPW = K * C
EP = NW * EPW
RPT = NP // NS
F32 = jnp.float32


def _sc_aggregate(g, src3, dst3, d):
    mesh = plsc.VectorSubcoreMesh(core_axis_name="c", subcore_axis_name="s")

    @functools.partial(
        pl.kernel,
        out_type=jax.ShapeDtypeStruct((NC, NP, d), F32),
        mesh=mesh,
        scratch_types=[
            pltpu.VMEM((KC, C), jnp.int32),
            pltpu.VMEM((KC, C), jnp.int32),
            pltpu.VMEM((C, d), F32),
            pltpu.VMEM((C, d), F32),
            pltpu.VMEM_SHARED((NP, d), F32),
            pltpu.SemaphoreType.DMA,
            pltpu.SemaphoreType.DMA,
            pltpu.SemaphoreType.DMA,
            pltpu.SemaphoreType.DMA,
        ],
    )
    def k(g_hbm, src_hbm, dst_hbm, out_hbm,
          src_v, dst_v, buf0, buf1, acc, sg0, sg1, ss0, ss1):
        c = lax.axis_index("c")
        s = lax.axis_index("s")
        wid = s * NC + c
        r0 = s * RPT
        chunks = [(i * C, C) for i in range(RPT // C)] + (
            [((RPT // C) * C, RPT % C)] if RPT % C else [])
        for off, sz in chunks:
            pltpu.sync_copy(g_hbm.at[pl.ds(r0 + off, sz)], buf0.at[pl.ds(0, sz)])
            pltpu.sync_copy(buf0.at[pl.ds(0, sz)], acc.at[pl.ds(r0 + off, sz)])
        plsc.subcore_barrier()

        bufs = (buf0, buf1)
        sgs = (sg0, sg1)
        sss = (ss0, ss1)

        for seg in range(K // KC):
            pltpu.sync_copy(src_hbm.at[wid, pl.ds(seg * KC, KC)], src_v)
            pltpu.sync_copy(dst_hbm.at[wid, pl.ds(seg * KC, KC)], dst_v)
            pltpu.async_copy(g_hbm.at[src_v.at[0]], buf0, sg0)

            def outer(j2, _):
                for b in range(2):
                    j = 2 * j2 + b
                    pltpu.make_async_copy(g_hbm.at[src_v.at[j]], bufs[b],
                                          sgs[b]).wait()
                    pltpu.async_copy(bufs[b], acc.at[dst_v.at[j]], sss[b],
                                     add=True)

                    @pl.when(j + 1 < KC)
                    def _():
                        @pl.when(j > 0)
                        def _():
                            pltpu.make_async_copy(
                                bufs[1 - b], acc.at[dst_v.at[j]],
                                sss[1 - b]).wait()
                        pltpu.async_copy(g_hbm.at[src_v.at[j + 1]], bufs[1 - b],
                                         sgs[1 - b])
                return 0

            lax.fori_loop(0, KC // 2, outer, 0)
            pltpu.make_async_copy(bufs[0], acc.at[dst_v.at[KC - 2]], sss[0]).wait()
            pltpu.make_async_copy(bufs[1], acc.at[dst_v.at[KC - 1]], sss[1]).wait()
        plsc.subcore_barrier()
        for off, sz in chunks:
            pltpu.sync_copy(acc.at[pl.ds(r0 + off, sz)], buf0.at[pl.ds(0, sz)])
            pltpu.sync_copy(buf0.at[pl.ds(0, sz)], out_hbm.at[c, pl.ds(r0 + off, sz)])

    return k(g, src3, dst3)


_R = 2528


def _dot(a, w):
    return lax.dot_general(a, w, (((1,), (0,)), ((), ())),
                           precision=lax.Precision.HIGHEST,
                           preferred_element_type=F32)


def _tc_prep(deg0, deg1, xpad, w1):
    def body(d0, d1, x, w, g_out, dv_out):
        dsum = d0[...][:, :1] + d1[...][:, :1] - 1.0
        dv = lax.rsqrt(dsum)
        dv_out[...] = dv
        g_out[...] = dv * _dot(x[...], w[...])

    return pl.pallas_call(
        body,
        grid=(NP // _R,),
        in_specs=[
            pl.BlockSpec((_R, 128), lambda i: (i, 0)),
            pl.BlockSpec((_R, 128), lambda i: (i, 0)),
            pl.BlockSpec((_R, 128), lambda i: (i, 0)),
            pl.BlockSpec((128, 128), lambda i: (0, 0)),
        ],
        out_specs=(
            pl.BlockSpec((_R, 128), lambda i: (i, 0)),
            pl.BlockSpec((_R, 1), lambda i: (i, 0)),
        ),
        out_shape=(
            jax.ShapeDtypeStruct((NP, 128), F32),
            jax.ShapeDtypeStruct((NP, 1), F32),
        ),
    )(deg0, deg1, xpad, w1)


def _tc_mid(p0, p1, g, dinv, b, w, d_out):
    def body(p0r, p1r, gr, dvr, br, wr, out):
        dv = dvr[...]
        z = dv * (p0r[...] + p1r[...] - gr[...]) + br[...]
        h = jnp.maximum(z, 0.0)
        out[...] = dv * _dot(h, wr[...])

    return pl.pallas_call(
        body,
        grid=(NP // _R,),
        in_specs=[
            pl.BlockSpec((_R, 128), lambda i: (i, 0)),
            pl.BlockSpec((_R, 128), lambda i: (i, 0)),
            pl.BlockSpec((_R, 128), lambda i: (i, 0)),
            pl.BlockSpec((_R, 1), lambda i: (i, 0)),
            pl.BlockSpec((1, 128), lambda i: (0, 0)),
            pl.BlockSpec((128, d_out), lambda i: (0, 0)),
        ],
        out_specs=pl.BlockSpec((_R, d_out), lambda i: (i, 0)),
        out_shape=jax.ShapeDtypeStruct((NP, d_out), F32),
    )(p0, p1, g, dinv, b, w)


def _tc_scale_relu(p0, p1, g, dinv, b):
    def body(p0r, p1r, gr, dvr, br, out):
        dv = dvr[...]
        z = dv * (p0r[...] + p1r[...] - gr[...]) + br[...]
        out[...] = dv * jnp.maximum(z, 0.0)

    return pl.pallas_call(
        body,
        grid=(NP // _R,),
        in_specs=[
            pl.BlockSpec((_R, 128), lambda i: (i, 0)),
            pl.BlockSpec((_R, 128), lambda i: (i, 0)),
            pl.BlockSpec((_R, 128), lambda i: (i, 0)),
            pl.BlockSpec((_R, 1), lambda i: (i, 0)),
            pl.BlockSpec((1, 128), lambda i: (0, 0)),
        ],
        out_specs=pl.BlockSpec((_R, 128), lambda i: (i, 0)),
        out_shape=jax.ShapeDtypeStruct((NP, 128), F32),
    )(p0, p1, g, dinv, b)


def _tc_final(p0, p1, u, dinv, w, b):
    def body(p0r, p1r, ur, dvr, wr, br, out):
        agg = dvr[...] * (p0r[...] + p1r[...] - ur[...])
        z = _dot(agg, wr[...]) + br[...]
        m = jnp.max(z, axis=1, keepdims=True)
        e = jnp.exp(z - m)
        lse = jnp.log(jnp.sum(e, axis=1, keepdims=True)) + m
        out[...] = z - lse

    return pl.pallas_call(
        body,
        grid=(NP // _R,),
        in_specs=[
            pl.BlockSpec((_R, 128), lambda i: (i, 0)),
            pl.BlockSpec((_R, 128), lambda i: (i, 0)),
            pl.BlockSpec((_R, 128), lambda i: (i, 0)),
            pl.BlockSpec((_R, 1), lambda i: (i, 0)),
            pl.BlockSpec((128, 16), lambda i: (0, 0)),
            pl.BlockSpec((1, 16), lambda i: (0, 0)),
        ],
        out_specs=pl.BlockSpec((_R, 16), lambda i: (i, 0)),
        out_shape=jax.ShapeDtypeStruct((NP, 16), F32),
    )(p0, p1, u, dinv, w, b)


def kernel(x, edge_index, W1, b1, W2, b2, W3, b3):
    src = edge_index[0].astype(jnp.int32)
    dst = edge_index[1].astype(jnp.int32)
    e = src.shape[0]
    pad = EP - e
    src3 = jnp.concatenate([src, jnp.full((pad,), N, jnp.int32)]).reshape(NW, K, C)
    dst3 = jnp.concatenate([dst, jnp.full((pad,), N, jnp.int32)]).reshape(NW, K, C)
    xpad = jnp.zeros((NP, x.shape[1]), F32).at[:N].set(x)

    degp = _sc_aggregate(jnp.ones((NP, 128), F32), src3, dst3, 128)
    g1, dinv = _tc_prep(degp[0], degp[1], xpad, W1)

    p1 = _sc_aggregate(g1, src3, dst3, 128)
    g2 = _tc_mid(p1[0], p1[1], g1, dinv, b1.reshape(1, -1), W2, 128)

    p2 = _sc_aggregate(g2, src3, dst3, 128)
    u = _tc_scale_relu(p2[0], p2[1], g2, dinv, b2.reshape(1, -1))

    p3 = _sc_aggregate(u, src3, dst3, 128)
    out = _tc_final(p3[0], p3[1], u, dinv, W3, b3.reshape(1, -1))
    return out[:N]


# speedup vs baseline: 18.9352x; 3.0919x over previous
import functools

import jax
import jax.numpy as jnp
from jax import lax
from jax.experimental import pallas as pl
from jax.experimental.pallas import tpu as pltpu
from jax.experimental.pallas import tpu_sc as plsc

N = 10000
NP = 10112
NC = 2
NS = 16
NW = NC * NS
C = 128
K = 80
KC = 40
EPW = K * C
EP = NW * EPW
RPT = NP // NS
F32 = jnp.float32


def _sc_aggregate(g, src3, dst3, d):
    mesh = plsc.VectorSubcoreMesh(core_axis_name="c", subcore_axis_name="s")

    @functools.partial(
        pl.kernel,
        out_type=jax.ShapeDtypeStruct((NC, NP, d), F32),
        mesh=mesh,
        scratch_types=[
            pltpu.VMEM((KC, C), jnp.int32),
            pltpu.VMEM((KC, C), jnp.int32),
            pltpu.VMEM((C, d), F32),
            pltpu.VMEM((C, d), F32),
            pltpu.VMEM_SHARED((NP, d), F32),
            pltpu.SemaphoreType.DMA,
            pltpu.SemaphoreType.DMA,
            pltpu.SemaphoreType.DMA,
            pltpu.SemaphoreType.DMA,
        ],
    )
    def k(g_hbm, src_hbm, dst_hbm, out_hbm,
          src_v, dst_v, buf0, buf1, acc, sg0, sg1, ss0, ss1):
        c = lax.axis_index("c")
        s = lax.axis_index("s")
        wid = s * NC + c
        r0 = s * RPT
        chunks = [(i * C, C) for i in range(RPT // C)] + (
            [((RPT // C) * C, RPT % C)] if RPT % C else [])
        for off, sz in chunks:
            pltpu.sync_copy(g_hbm.at[pl.ds(r0 + off, sz)], buf0.at[pl.ds(0, sz)])
            pltpu.sync_copy(buf0.at[pl.ds(0, sz)], acc.at[pl.ds(r0 + off, sz)])
        plsc.subcore_barrier()

        bufs = (buf0, buf1)
        sgs = (sg0, sg1)
        sss = (ss0, ss1)

        for seg in range(K // KC):
            pltpu.sync_copy(src_hbm.at[wid, pl.ds(seg * KC, KC)], src_v)
            pltpu.sync_copy(dst_hbm.at[wid, pl.ds(seg * KC, KC)], dst_v)
            pltpu.async_copy(g_hbm.at[src_v.at[0]], buf0, sg0)

            def outer(j2, _):
                for b in range(2):
                    j = 2 * j2 + b
                    pltpu.make_async_copy(g_hbm.at[src_v.at[j]], bufs[b],
                                          sgs[b]).wait()
                    pltpu.async_copy(bufs[b], acc.at[dst_v.at[j]], sss[b],
                                     add=True)

                    @pl.when(j + 1 < KC)
                    def _():
                        @pl.when(j > 0)
                        def _():
                            pltpu.make_async_copy(
                                bufs[1 - b], acc.at[dst_v.at[j]],
                                sss[1 - b]).wait()
                        pltpu.async_copy(g_hbm.at[src_v.at[j + 1]], bufs[1 - b],
                                         sgs[1 - b])
                return 0

            lax.fori_loop(0, KC // 2, outer, 0)
            pltpu.make_async_copy(bufs[0], acc.at[dst_v.at[KC - 2]], sss[0]).wait()
            pltpu.make_async_copy(bufs[1], acc.at[dst_v.at[KC - 1]], sss[1]).wait()
        plsc.subcore_barrier()
        for off, sz in chunks:
            pltpu.sync_copy(acc.at[pl.ds(r0 + off, sz)], buf0.at[pl.ds(0, sz)])
            pltpu.sync_copy(buf0.at[pl.ds(0, sz)], out_hbm.at[c, pl.ds(r0 + off, sz)])

    return k(g, src3, dst3)


_R = 2528


def _dot(a, w):
    return lax.dot_general(a, w, (((1,), (0,)), ((), ())),
                           precision=lax.Precision.HIGHEST,
                           preferred_element_type=F32)


def _tc_prep(deg0, deg1, xpad, w1):
    def body(d0, d1, x, w, g_out, dv_out):
        dsum = d0[...][:, :1] + d1[...][:, :1] - 1.0
        dv = lax.rsqrt(dsum)
        dv_out[...] = dv
        g_out[...] = dv * _dot(x[...], w[...])

    return pl.pallas_call(
        body,
        grid=(NP // _R,),
        in_specs=[
            pl.BlockSpec((_R, 128), lambda i: (i, 0)),
            pl.BlockSpec((_R, 128), lambda i: (i, 0)),
            pl.BlockSpec((_R, 128), lambda i: (i, 0)),
            pl.BlockSpec((128, 128), lambda i: (0, 0)),
        ],
        out_specs=(
            pl.BlockSpec((_R, 128), lambda i: (i, 0)),
            pl.BlockSpec((_R, 1), lambda i: (i, 0)),
        ),
        out_shape=(
            jax.ShapeDtypeStruct((NP, 128), F32),
            jax.ShapeDtypeStruct((NP, 1), F32),
        ),
    )(deg0, deg1, xpad, w1)


def _tc_mid(p0, p1, g, dinv, b, w, d_out):
    def body(p0r, p1r, gr, dvr, br, wr, out):
        dv = dvr[...]
        z = dv * (p0r[...] + p1r[...] - gr[...]) + br[...]
        h = jnp.maximum(z, 0.0)
        out[...] = dv * _dot(h, wr[...])

    return pl.pallas_call(
        body,
        grid=(NP // _R,),
        in_specs=[
            pl.BlockSpec((_R, 128), lambda i: (i, 0)),
            pl.BlockSpec((_R, 128), lambda i: (i, 0)),
            pl.BlockSpec((_R, 128), lambda i: (i, 0)),
            pl.BlockSpec((_R, 1), lambda i: (i, 0)),
            pl.BlockSpec((1, 128), lambda i: (0, 0)),
            pl.BlockSpec((128, d_out), lambda i: (0, 0)),
        ],
        out_specs=pl.BlockSpec((_R, d_out), lambda i: (i, 0)),
        out_shape=jax.ShapeDtypeStruct((NP, d_out), F32),
    )(p0, p1, g, dinv, b, w)


def _tc_scale_relu(p0, p1, g, dinv, b):
    def body(p0r, p1r, gr, dvr, br, out):
        dv = dvr[...]
        z = dv * (p0r[...] + p1r[...] - gr[...]) + br[...]
        out[...] = dv * jnp.maximum(z, 0.0)

    return pl.pallas_call(
        body,
        grid=(NP // _R,),
        in_specs=[
            pl.BlockSpec((_R, 128), lambda i: (i, 0)),
            pl.BlockSpec((_R, 128), lambda i: (i, 0)),
            pl.BlockSpec((_R, 128), lambda i: (i, 0)),
            pl.BlockSpec((_R, 1), lambda i: (i, 0)),
            pl.BlockSpec((1, 128), lambda i: (0, 0)),
        ],
        out_specs=pl.BlockSpec((_R, 128), lambda i: (i, 0)),
        out_shape=jax.ShapeDtypeStruct((NP, 128), F32),
    )(p0, p1, g, dinv, b)


def _tc_final(p0, p1, u, dinv, w, b):
    def body(p0r, p1r, ur, dvr, wr, br, out):
        agg = dvr[...] * (p0r[...] + p1r[...] - ur[...])
        z = _dot(agg, wr[...]) + br[...]
        m = jnp.max(z, axis=1, keepdims=True)
        e = jnp.exp(z - m)
        lse = jnp.log(jnp.sum(e, axis=1, keepdims=True)) + m
        out[...] = z - lse

    return pl.pallas_call(
        body,
        grid=(NP // _R,),
        in_specs=[
            pl.BlockSpec((_R, 128), lambda i: (i, 0)),
            pl.BlockSpec((_R, 128), lambda i: (i, 0)),
            pl.BlockSpec((_R, 128), lambda i: (i, 0)),
            pl.BlockSpec((_R, 1), lambda i: (i, 0)),
            pl.BlockSpec((128, 16), lambda i: (0, 0)),
            pl.BlockSpec((1, 16), lambda i: (0, 0)),
        ],
        out_specs=pl.BlockSpec((_R, 16), lambda i: (i, 0)),
        out_shape=jax.ShapeDtypeStruct((NP, 16), F32),
    )(p0, p1, u, dinv, w, b)


def kernel(x, edge_index, W1, b1, W2, b2, W3, b3):
    src = edge_index[0].astype(jnp.int32)
    dst = edge_index[1].astype(jnp.int32)
    e = src.shape[0]
    pad = EP - e
    trash = N + (jnp.arange(pad, dtype=jnp.int32) % (NP - N))
    src3 = jnp.concatenate([src, trash]).reshape(NW, K, C)
    dst3 = jnp.concatenate([dst, trash]).reshape(NW, K, C)
    xpad = jnp.zeros((NP, x.shape[1]), F32).at[:N].set(x)

    degp = _sc_aggregate(jnp.ones((NP, 128), F32), src3, dst3, 128)
    g1, dinv = _tc_prep(degp[0], degp[1], xpad, W1)

    p1 = _sc_aggregate(g1, src3, dst3, 128)
    g2 = _tc_mid(p1[0], p1[1], g1, dinv, b1.reshape(1, -1), W2, 128)

    p2 = _sc_aggregate(g2, src3, dst3, 128)
    u = _tc_scale_relu(p2[0], p2[1], g2, dinv, b2.reshape(1, -1))

    p3 = _sc_aggregate(u, src3, dst3, 128)
    out = _tc_final(p3[0], p3[1], u, dinv, W3, b3.reshape(1, -1))
    return out[:N]


# speedup vs baseline: 20.8743x; 1.1024x over previous
import functools

import jax
import jax.numpy as jnp
from jax import lax
from jax.experimental import pallas as pl
from jax.experimental.pallas import tpu as pltpu
from jax.experimental.pallas import tpu_sc as plsc

N = 10000
NP = 10112
NC = 2
NS = 16
NW = NC * NS
C = 128
K = 80
KC = 40
EPW = K * C
EP = NW * EPW
RPT = NP // NS
F32 = jnp.float32


def _sc_degree(ones, dst3):
    mesh = plsc.VectorSubcoreMesh(core_axis_name="c", subcore_axis_name="s")

    @functools.partial(
        pl.kernel,
        out_type=jax.ShapeDtypeStruct((NC, NP, 128), F32),
        mesh=mesh,
        scratch_types=[
            pltpu.VMEM((KC, C), jnp.int32),
            pltpu.VMEM((C, 128), F32),
            pltpu.VMEM_SHARED((NP, 128), F32),
            pltpu.SemaphoreType.DMA,
            pltpu.SemaphoreType.DMA,
        ],
    )
    def k(ones_hbm, dst_hbm, out_hbm, dst_v, buf, acc, ss0, ss1):
        c = lax.axis_index("c")
        s = lax.axis_index("s")
        wid = s * NC + c
        r0 = s * RPT
        chunks = [(i * C, C) for i in range(RPT // C)] + (
            [((RPT // C) * C, RPT % C)] if RPT % C else [])
        pltpu.sync_copy(ones_hbm.at[pl.ds(0, C)], buf)
        for off, sz in chunks:
            pltpu.sync_copy(buf.at[pl.ds(0, sz)], acc.at[pl.ds(r0 + off, sz)])
        plsc.subcore_barrier()

        sss = (ss0, ss1)
        for seg in range(K // KC):
            pltpu.sync_copy(dst_hbm.at[wid, pl.ds(seg * KC, KC)], dst_v)
            pltpu.async_copy(buf, acc.at[dst_v.at[0]], ss0, add=True)
            pltpu.async_copy(buf, acc.at[dst_v.at[1]], ss1, add=True)

            def outer(j2, _):
                for b in range(2):
                    j = 2 * j2 + b
                    pltpu.make_async_copy(buf, acc.at[dst_v.at[j]],
                                          sss[b]).wait()
                    pltpu.async_copy(buf, acc.at[dst_v.at[j]], sss[b],
                                     add=True)
                return 0

            lax.fori_loop(1, KC // 2, outer, 0)
            pltpu.make_async_copy(buf, acc.at[dst_v.at[KC - 2]], sss[0]).wait()
            pltpu.make_async_copy(buf, acc.at[dst_v.at[KC - 1]], sss[1]).wait()
        plsc.subcore_barrier()
        for off, sz in chunks:
            pltpu.sync_copy(acc.at[pl.ds(r0 + off, sz)], buf.at[pl.ds(0, sz)])
            pltpu.sync_copy(buf.at[pl.ds(0, sz)], out_hbm.at[c, pl.ds(r0 + off, sz)])

    return k(ones, dst3)


def _sc_aggregate(g, src3, dst3, d):
    mesh = plsc.VectorSubcoreMesh(core_axis_name="c", subcore_axis_name="s")

    @functools.partial(
        pl.kernel,
        out_type=jax.ShapeDtypeStruct((NC, NP, d), F32),
        mesh=mesh,
        scratch_types=[
            pltpu.VMEM((KC, C), jnp.int32),
            pltpu.VMEM((KC, C), jnp.int32),
            pltpu.VMEM((C, d), F32),
            pltpu.VMEM((C, d), F32),
            pltpu.VMEM_SHARED((NP, d), F32),
            pltpu.SemaphoreType.DMA,
            pltpu.SemaphoreType.DMA,
            pltpu.SemaphoreType.DMA,
            pltpu.SemaphoreType.DMA,
        ],
    )
    def k(g_hbm, src_hbm, dst_hbm, out_hbm,
          src_v, dst_v, buf0, buf1, acc, sg0, sg1, ss0, ss1):
        c = lax.axis_index("c")
        s = lax.axis_index("s")
        wid = s * NC + c
        r0 = s * RPT
        chunks = [(i * C, C) for i in range(RPT // C)] + (
            [((RPT // C) * C, RPT % C)] if RPT % C else [])
        for off, sz in chunks:
            pltpu.sync_copy(g_hbm.at[pl.ds(r0 + off, sz)], buf0.at[pl.ds(0, sz)])
            pltpu.sync_copy(buf0.at[pl.ds(0, sz)], acc.at[pl.ds(r0 + off, sz)])
        plsc.subcore_barrier()

        bufs = (buf0, buf1)
        sgs = (sg0, sg1)
        sss = (ss0, ss1)

        for seg in range(K // KC):
            pltpu.sync_copy(src_hbm.at[wid, pl.ds(seg * KC, KC)], src_v)
            pltpu.sync_copy(dst_hbm.at[wid, pl.ds(seg * KC, KC)], dst_v)
            pltpu.async_copy(g_hbm.at[src_v.at[0]], buf0, sg0)

            def outer(j2, _):
                for b in range(2):
                    j = 2 * j2 + b
                    pltpu.make_async_copy(g_hbm.at[src_v.at[j]], bufs[b],
                                          sgs[b]).wait()
                    pltpu.async_copy(bufs[b], acc.at[dst_v.at[j]], sss[b],
                                     add=True)

                    @pl.when(j + 1 < KC)
                    def _():
                        @pl.when(j > 0)
                        def _():
                            pltpu.make_async_copy(
                                bufs[1 - b], acc.at[dst_v.at[j]],
                                sss[1 - b]).wait()
                        pltpu.async_copy(g_hbm.at[src_v.at[j + 1]], bufs[1 - b],
                                         sgs[1 - b])
                return 0

            lax.fori_loop(0, KC // 2, outer, 0)
            pltpu.make_async_copy(bufs[0], acc.at[dst_v.at[KC - 2]], sss[0]).wait()
            pltpu.make_async_copy(bufs[1], acc.at[dst_v.at[KC - 1]], sss[1]).wait()
        plsc.subcore_barrier()
        for off, sz in chunks:
            pltpu.sync_copy(acc.at[pl.ds(r0 + off, sz)], buf0.at[pl.ds(0, sz)])
            pltpu.sync_copy(buf0.at[pl.ds(0, sz)], out_hbm.at[c, pl.ds(r0 + off, sz)])

    return k(g, src3, dst3)


_R = 2528


def _dot(a, w):
    return lax.dot_general(a, w, (((1,), (0,)), ((), ())),
                           precision=lax.Precision.HIGHEST,
                           preferred_element_type=F32)


def _tc_prep(deg0, deg1, xpad, w1):
    def body(d0, d1, x, w, g_out, dv_out):
        dsum = d0[...][:, :1] + d1[...][:, :1] - 1.0
        dv = lax.rsqrt(dsum)
        dv_out[...] = dv
        g_out[...] = dv * _dot(x[...], w[...])

    return pl.pallas_call(
        body,
        grid=(NP // _R,),
        in_specs=[
            pl.BlockSpec((_R, 128), lambda i: (i, 0)),
            pl.BlockSpec((_R, 128), lambda i: (i, 0)),
            pl.BlockSpec((_R, 128), lambda i: (i, 0)),
            pl.BlockSpec((128, 128), lambda i: (0, 0)),
        ],
        out_specs=(
            pl.BlockSpec((_R, 128), lambda i: (i, 0)),
            pl.BlockSpec((_R, 1), lambda i: (i, 0)),
        ),
        out_shape=(
            jax.ShapeDtypeStruct((NP, 128), F32),
            jax.ShapeDtypeStruct((NP, 1), F32),
        ),
    )(deg0, deg1, xpad, w1)


def _tc_mid(p0, p1, g, dinv, b, w, d_out):
    def body(p0r, p1r, gr, dvr, br, wr, out):
        dv = dvr[...]
        z = dv * (p0r[...] + p1r[...] - gr[...]) + br[...]
        h = jnp.maximum(z, 0.0)
        out[...] = dv * _dot(h, wr[...])

    return pl.pallas_call(
        body,
        grid=(NP // _R,),
        in_specs=[
            pl.BlockSpec((_R, 128), lambda i: (i, 0)),
            pl.BlockSpec((_R, 128), lambda i: (i, 0)),
            pl.BlockSpec((_R, 128), lambda i: (i, 0)),
            pl.BlockSpec((_R, 1), lambda i: (i, 0)),
            pl.BlockSpec((1, 128), lambda i: (0, 0)),
            pl.BlockSpec((128, d_out), lambda i: (0, 0)),
        ],
        out_specs=pl.BlockSpec((_R, d_out), lambda i: (i, 0)),
        out_shape=jax.ShapeDtypeStruct((NP, d_out), F32),
    )(p0, p1, g, dinv, b, w)


def _tc_scale_relu(p0, p1, g, dinv, b):
    def body(p0r, p1r, gr, dvr, br, out):
        dv = dvr[...]
        z = dv * (p0r[...] + p1r[...] - gr[...]) + br[...]
        out[...] = dv * jnp.maximum(z, 0.0)

    return pl.pallas_call(
        body,
        grid=(NP // _R,),
        in_specs=[
            pl.BlockSpec((_R, 128), lambda i: (i, 0)),
            pl.BlockSpec((_R, 128), lambda i: (i, 0)),
            pl.BlockSpec((_R, 128), lambda i: (i, 0)),
            pl.BlockSpec((_R, 1), lambda i: (i, 0)),
            pl.BlockSpec((1, 128), lambda i: (0, 0)),
        ],
        out_specs=pl.BlockSpec((_R, 128), lambda i: (i, 0)),
        out_shape=jax.ShapeDtypeStruct((NP, 128), F32),
    )(p0, p1, g, dinv, b)


def _tc_final(p0, p1, u, dinv, w, b):
    def body(p0r, p1r, ur, dvr, wr, br, out):
        agg = dvr[...] * (p0r[...] + p1r[...] - ur[...])
        z = _dot(agg, wr[...]) + br[...]
        m = jnp.max(z, axis=1, keepdims=True)
        e = jnp.exp(z - m)
        lse = jnp.log(jnp.sum(e, axis=1, keepdims=True)) + m
        out[...] = z - lse

    return pl.pallas_call(
        body,
        grid=(NP // _R,),
        in_specs=[
            pl.BlockSpec((_R, 128), lambda i: (i, 0)),
            pl.BlockSpec((_R, 128), lambda i: (i, 0)),
            pl.BlockSpec((_R, 128), lambda i: (i, 0)),
            pl.BlockSpec((_R, 1), lambda i: (i, 0)),
            pl.BlockSpec((128, 16), lambda i: (0, 0)),
            pl.BlockSpec((1, 16), lambda i: (0, 0)),
        ],
        out_specs=pl.BlockSpec((_R, 16), lambda i: (i, 0)),
        out_shape=jax.ShapeDtypeStruct((NP, 16), F32),
    )(p0, p1, u, dinv, w, b)


def kernel(x, edge_index, W1, b1, W2, b2, W3, b3):
    src = edge_index[0].astype(jnp.int32)
    dst = edge_index[1].astype(jnp.int32)
    e = src.shape[0]
    pad = EP - e
    trash = N + (jnp.arange(pad, dtype=jnp.int32) % (NP - N))
    src3 = jnp.concatenate([src, trash]).reshape(NW, K, C)
    dst3 = jnp.concatenate([dst, trash]).reshape(NW, K, C)
    xpad = jnp.zeros((NP, x.shape[1]), F32).at[:N].set(x)

    degp = _sc_degree(jnp.ones((NP, 128), F32), dst3)
    g1, dinv = _tc_prep(degp[0], degp[1], xpad, W1)

    p1 = _sc_aggregate(g1, src3, dst3, 128)
    g2 = _tc_mid(p1[0], p1[1], g1, dinv, b1.reshape(1, -1), W2, 128)

    p2 = _sc_aggregate(g2, src3, dst3, 128)
    u = _tc_scale_relu(p2[0], p2[1], g2, dinv, b2.reshape(1, -1))

    p3 = _sc_aggregate(u, src3, dst3, 128)
    out = _tc_final(p3[0], p3[1], u, dinv, W3, b3.reshape(1, -1))
    return out[:N]


# speedup vs baseline: 21.4094x; 1.0256x over previous
import functools

import jax
import jax.numpy as jnp
from jax import lax
from jax.experimental import pallas as pl
from jax.experimental.pallas import tpu as pltpu
from jax.experimental.pallas import tpu_sc as plsc

N = 10000
NP = 10112
NC = 2
NS = 16
NW = NC * NS
C = 128
K = 80
KC = 40
EPW = K * C
EP = NW * EPW
RPT = NP // NS
F32 = jnp.float32


def _sc_degree(ones, dst3):
    mesh = plsc.VectorSubcoreMesh(core_axis_name="c", subcore_axis_name="s")

    @functools.partial(
        pl.kernel,
        out_type=jax.ShapeDtypeStruct((NC, NP, 128), F32),
        mesh=mesh,
        scratch_types=[
            pltpu.VMEM((KC, C), jnp.int32),
            pltpu.VMEM((C, 128), F32),
            pltpu.VMEM_SHARED((NP, 128), F32),
            pltpu.SemaphoreType.DMA,
            pltpu.SemaphoreType.DMA,
        ],
    )
    def k(ones_hbm, dst_hbm, out_hbm, dst_v, buf, acc, ss0, ss1):
        c = lax.axis_index("c")
        s = lax.axis_index("s")
        wid = s * NC + c
        r0 = s * RPT
        chunks = [(i * C, C) for i in range(RPT // C)] + (
            [((RPT // C) * C, RPT % C)] if RPT % C else [])
        pltpu.sync_copy(ones_hbm.at[pl.ds(0, C)], buf)
        for off, sz in chunks:
            pltpu.sync_copy(buf.at[pl.ds(0, sz)], acc.at[pl.ds(r0 + off, sz)])
        plsc.subcore_barrier()

        sss = (ss0, ss1)
        for seg in range(K // KC):
            pltpu.sync_copy(dst_hbm.at[wid, pl.ds(seg * KC, KC)], dst_v)
            pltpu.async_copy(buf, acc.at[dst_v.at[0]], ss0, add=True)
            pltpu.async_copy(buf, acc.at[dst_v.at[1]], ss1, add=True)

            def outer(j2, _):
                for b in range(2):
                    j = 2 * j2 + b
                    pltpu.make_async_copy(buf, acc.at[dst_v.at[j]],
                                          sss[b]).wait()
                    pltpu.async_copy(buf, acc.at[dst_v.at[j]], sss[b],
                                     add=True)
                return 0

            lax.fori_loop(1, KC // 2, outer, 0)
            pltpu.make_async_copy(buf, acc.at[dst_v.at[KC - 2]], sss[0]).wait()
            pltpu.make_async_copy(buf, acc.at[dst_v.at[KC - 1]], sss[1]).wait()
        plsc.subcore_barrier()
        for off, sz in chunks:
            pltpu.sync_copy(acc.at[pl.ds(r0 + off, sz)], buf.at[pl.ds(0, sz)])
            pltpu.sync_copy(buf.at[pl.ds(0, sz)], out_hbm.at[c, pl.ds(r0 + off, sz)])

    return k(ones, dst3)


def _sc_aggregate(g, src3, dst3, d):
    mesh = plsc.VectorSubcoreMesh(core_axis_name="c", subcore_axis_name="s")

    @functools.partial(
        pl.kernel,
        out_type=jax.ShapeDtypeStruct((NC, NP, d), F32),
        mesh=mesh,
        scratch_types=[
            pltpu.VMEM((KC, C), jnp.int32),
            pltpu.VMEM((KC, C), jnp.int32),
            pltpu.VMEM((C, d), F32),
            pltpu.VMEM((C, d), F32),
            pltpu.VMEM_SHARED((NP, d), F32),
            pltpu.SemaphoreType.DMA,
            pltpu.SemaphoreType.DMA,
            pltpu.SemaphoreType.DMA,
            pltpu.SemaphoreType.DMA,
        ],
    )
    def k(g_hbm, src_hbm, dst_hbm, out_hbm,
          src_v, dst_v, buf0, buf1, acc, sg0, sg1, ss0, ss1):
        c = lax.axis_index("c")
        s = lax.axis_index("s")
        wid = s * NC + c
        r0 = s * RPT
        bufs = (buf0, buf1)
        sgs = (sg0, sg1)
        sss = (ss0, ss1)
        chunks = [(i * C, C) for i in range(RPT // C)] + (
            [((RPT // C) * C, RPT % C)] if RPT % C else [])
        nch = len(chunks)
        pltpu.async_copy(g_hbm.at[pl.ds(r0 + chunks[0][0], chunks[0][1])],
                         bufs[0].at[pl.ds(0, chunks[0][1])], sgs[0])
        for i, (off, sz) in enumerate(chunks):
            b = i % 2
            pltpu.make_async_copy(g_hbm.at[pl.ds(r0 + off, sz)],
                                  bufs[b].at[pl.ds(0, sz)], sgs[b]).wait()
            if i + 1 < nch:
                noff, nsz = chunks[i + 1]
                if i >= 1:
                    poff, psz = chunks[i - 1]
                    pltpu.make_async_copy(
                        bufs[1 - b].at[pl.ds(0, psz)],
                        acc.at[pl.ds(r0 + poff, psz)], sss[1 - b]).wait()
                pltpu.async_copy(g_hbm.at[pl.ds(r0 + noff, nsz)],
                                 bufs[1 - b].at[pl.ds(0, nsz)], sgs[1 - b])
            pltpu.async_copy(bufs[b].at[pl.ds(0, sz)],
                             acc.at[pl.ds(r0 + off, sz)], sss[b])
        for i in (nch - 2, nch - 1):
            off, sz = chunks[i]
            pltpu.make_async_copy(bufs[i % 2].at[pl.ds(0, sz)],
                                  acc.at[pl.ds(r0 + off, sz)], sss[i % 2]).wait()
        plsc.subcore_barrier()

        for seg in range(K // KC):
            pltpu.sync_copy(src_hbm.at[wid, pl.ds(seg * KC, KC)], src_v)
            pltpu.sync_copy(dst_hbm.at[wid, pl.ds(seg * KC, KC)], dst_v)
            pltpu.async_copy(g_hbm.at[src_v.at[0]], buf0, sg0)

            def outer(j2, _):
                for b in range(2):
                    j = 2 * j2 + b
                    pltpu.make_async_copy(g_hbm.at[src_v.at[j]], bufs[b],
                                          sgs[b]).wait()
                    pltpu.async_copy(bufs[b], acc.at[dst_v.at[j]], sss[b],
                                     add=True)

                    @pl.when(j + 1 < KC)
                    def _():
                        @pl.when(j > 0)
                        def _():
                            pltpu.make_async_copy(
                                bufs[1 - b], acc.at[dst_v.at[j]],
                                sss[1 - b]).wait()
                        pltpu.async_copy(g_hbm.at[src_v.at[j + 1]], bufs[1 - b],
                                         sgs[1 - b])
                return 0

            lax.fori_loop(0, KC // 2, outer, 0)
            pltpu.make_async_copy(bufs[0], acc.at[dst_v.at[KC - 2]], sss[0]).wait()
            pltpu.make_async_copy(bufs[1], acc.at[dst_v.at[KC - 1]], sss[1]).wait()
        plsc.subcore_barrier()
        pltpu.async_copy(acc.at[pl.ds(r0 + chunks[0][0], chunks[0][1])],
                         bufs[0].at[pl.ds(0, chunks[0][1])], sgs[0])
        for i, (off, sz) in enumerate(chunks):
            b = i % 2
            pltpu.make_async_copy(acc.at[pl.ds(r0 + off, sz)],
                                  bufs[b].at[pl.ds(0, sz)], sgs[b]).wait()
            if i + 1 < nch:
                noff, nsz = chunks[i + 1]
                if i >= 1:
                    poff, psz = chunks[i - 1]
                    pltpu.make_async_copy(
                        bufs[1 - b].at[pl.ds(0, psz)],
                        out_hbm.at[c, pl.ds(r0 + poff, psz)], sss[1 - b]).wait()
                pltpu.async_copy(acc.at[pl.ds(r0 + noff, nsz)],
                                 bufs[1 - b].at[pl.ds(0, nsz)], sgs[1 - b])
            pltpu.async_copy(bufs[b].at[pl.ds(0, sz)],
                             out_hbm.at[c, pl.ds(r0 + off, sz)], sss[b])
        for i in (nch - 2, nch - 1):
            off, sz = chunks[i]
            pltpu.make_async_copy(bufs[i % 2].at[pl.ds(0, sz)],
                                  out_hbm.at[c, pl.ds(r0 + off, sz)],
                                  sss[i % 2]).wait()

    return k(g, src3, dst3)


_R = 2528


def _dot(a, w):
    return lax.dot_general(a, w, (((1,), (0,)), ((), ())),
                           precision=lax.Precision.HIGHEST,
                           preferred_element_type=F32)


def _tc_prep(deg0, deg1, xpad, w1):
    def body(d0, d1, x, w, g_out, dv_out):
        dsum = d0[...][:, :1] + d1[...][:, :1] - 1.0
        dv = lax.rsqrt(dsum)
        dv_out[...] = dv
        g_out[...] = dv * _dot(x[...], w[...])

    return pl.pallas_call(
        body,
        grid=(NP // _R,),
        in_specs=[
            pl.BlockSpec((_R, 128), lambda i: (i, 0)),
            pl.BlockSpec((_R, 128), lambda i: (i, 0)),
            pl.BlockSpec((_R, 128), lambda i: (i, 0)),
            pl.BlockSpec((128, 128), lambda i: (0, 0)),
        ],
        out_specs=(
            pl.BlockSpec((_R, 128), lambda i: (i, 0)),
            pl.BlockSpec((_R, 1), lambda i: (i, 0)),
        ),
        out_shape=(
            jax.ShapeDtypeStruct((NP, 128), F32),
            jax.ShapeDtypeStruct((NP, 1), F32),
        ),
    )(deg0, deg1, xpad, w1)


def _tc_mid(p0, p1, g, dinv, b, w, d_out):
    def body(p0r, p1r, gr, dvr, br, wr, out):
        dv = dvr[...]
        z = dv * (p0r[...] + p1r[...] - gr[...]) + br[...]
        h = jnp.maximum(z, 0.0)
        out[...] = dv * _dot(h, wr[...])

    return pl.pallas_call(
        body,
        grid=(NP // _R,),
        in_specs=[
            pl.BlockSpec((_R, 128), lambda i: (i, 0)),
            pl.BlockSpec((_R, 128), lambda i: (i, 0)),
            pl.BlockSpec((_R, 128), lambda i: (i, 0)),
            pl.BlockSpec((_R, 1), lambda i: (i, 0)),
            pl.BlockSpec((1, 128), lambda i: (0, 0)),
            pl.BlockSpec((128, d_out), lambda i: (0, 0)),
        ],
        out_specs=pl.BlockSpec((_R, d_out), lambda i: (i, 0)),
        out_shape=jax.ShapeDtypeStruct((NP, d_out), F32),
    )(p0, p1, g, dinv, b, w)


def _tc_scale_relu(p0, p1, g, dinv, b):
    def body(p0r, p1r, gr, dvr, br, out):
        dv = dvr[...]
        z = dv * (p0r[...] + p1r[...] - gr[...]) + br[...]
        out[...] = dv * jnp.maximum(z, 0.0)

    return pl.pallas_call(
        body,
        grid=(NP // _R,),
        in_specs=[
            pl.BlockSpec((_R, 128), lambda i: (i, 0)),
            pl.BlockSpec((_R, 128), lambda i: (i, 0)),
            pl.BlockSpec((_R, 128), lambda i: (i, 0)),
            pl.BlockSpec((_R, 1), lambda i: (i, 0)),
            pl.BlockSpec((1, 128), lambda i: (0, 0)),
        ],
        out_specs=pl.BlockSpec((_R, 128), lambda i: (i, 0)),
        out_shape=jax.ShapeDtypeStruct((NP, 128), F32),
    )(p0, p1, g, dinv, b)


def _tc_final(p0, p1, u, dinv, w, b):
    def body(p0r, p1r, ur, dvr, wr, br, out):
        agg = dvr[...] * (p0r[...] + p1r[...] - ur[...])
        z = _dot(agg, wr[...]) + br[...]
        m = jnp.max(z, axis=1, keepdims=True)
        e = jnp.exp(z - m)
        lse = jnp.log(jnp.sum(e, axis=1, keepdims=True)) + m
        out[...] = z - lse

    return pl.pallas_call(
        body,
        grid=(NP // _R,),
        in_specs=[
            pl.BlockSpec((_R, 128), lambda i: (i, 0)),
            pl.BlockSpec((_R, 128), lambda i: (i, 0)),
            pl.BlockSpec((_R, 128), lambda i: (i, 0)),
            pl.BlockSpec((_R, 1), lambda i: (i, 0)),
            pl.BlockSpec((128, 16), lambda i: (0, 0)),
            pl.BlockSpec((1, 16), lambda i: (0, 0)),
        ],
        out_specs=pl.BlockSpec((_R, 16), lambda i: (i, 0)),
        out_shape=jax.ShapeDtypeStruct((NP, 16), F32),
    )(p0, p1, u, dinv, w, b)


def kernel(x, edge_index, W1, b1, W2, b2, W3, b3):
    src = edge_index[0].astype(jnp.int32)
    dst = edge_index[1].astype(jnp.int32)
    e = src.shape[0]
    pad = EP - e
    trash = N + (jnp.arange(pad, dtype=jnp.int32) % (NP - N))
    src3 = jnp.concatenate([src, trash]).reshape(NW, K, C)
    dst3 = jnp.concatenate([dst, trash]).reshape(NW, K, C)
    xpad = jnp.zeros((NP, x.shape[1]), F32).at[:N].set(x)

    degp = _sc_degree(jnp.ones((NP, 128), F32), dst3)
    g1, dinv = _tc_prep(degp[0], degp[1], xpad, W1)

    p1 = _sc_aggregate(g1, src3, dst3, 128)
    g2 = _tc_mid(p1[0], p1[1], g1, dinv, b1.reshape(1, -1), W2, 128)

    p2 = _sc_aggregate(g2, src3, dst3, 128)
    u = _tc_scale_relu(p2[0], p2[1], g2, dinv, b2.reshape(1, -1))

    p3 = _sc_aggregate(u, src3, dst3, 128)
    out = _tc_final(p3[0], p3[1], u, dinv, W3, b3.reshape(1, -1))
    return out[:N]
